# baseline (speedup 1.0000x reference)
import functools

import jax
import jax.numpy as jnp
from jax.experimental import pallas as pl

_EPS = 1e-5


def _enc_body(xp_ref, w1t_ref, b1_ref, w2t_ref, b2_ref, wft_ref, wgt_ref,
              b3_ref, w4t_ref, b4_ref, z_ref):
    x = xp_ref[0]
    f1 = jnp.maximum(
        jnp.dot(x, w1t_ref[...], preferred_element_type=jnp.float32)
        + b1_ref[...], 0.0)
    f = (jnp.dot(f1, w2t_ref[...], preferred_element_type=jnp.float32)
         + b2_ref[...])
    g = jnp.max(f, axis=0, keepdims=True)
    g8 = jnp.broadcast_to(g, (8, 256))
    gterm = jnp.dot(g8, wgt_ref[...],
                    preferred_element_type=jnp.float32)[0:1]
    h1 = jnp.maximum(
        jnp.dot(f, wft_ref[...], preferred_element_type=jnp.float32)
        + gterm + b3_ref[...], 0.0)
    h = (jnp.dot(h1, w4t_ref[...], preferred_element_type=jnp.float32)
         + b4_ref[...])
    z_ref[0] = jnp.max(h, axis=0, keepdims=True)


def _encoder(xp, w1t, b1, w2t, b2, wft, wgt, b3, w4t, b4):
    B, N, _ = xp.shape
    C = w4t.shape[1]
    rep = lambda shape: pl.BlockSpec(shape, lambda b: (0,) * len(shape))
    return pl.pallas_call(
        _enc_body,
        grid=(B,),
        in_specs=[
            pl.BlockSpec((1, N, 8), lambda b: (b, 0, 0)),
            rep(w1t.shape), rep(b1.shape), rep(w2t.shape), rep(b2.shape),
            rep(wft.shape), rep(wgt.shape), rep(b3.shape), rep(w4t.shape),
            rep(b4.shape),
        ],
        out_specs=pl.BlockSpec((1, 1, C), lambda b: (b, 0, 0)),
        out_shape=jax.ShapeDtypeStruct((B, 1, C), jnp.float32),
    )(xp, w1t, b1, w2t, b2, wft, wgt, b3, w4t, b4).reshape(B, C)


def _mm_body(x_ref, w_ref, b_ref, o_ref, *, relu):
    y = jax.lax.dot_general(
        x_ref[...], w_ref[...], (((1,), (1,)), ((), ())),
        preferred_element_type=jnp.float32) + b_ref[...]
    if relu:
        y = jnp.maximum(y, 0.0)
    o_ref[...] = y


def _mm(x, w, b, relu, n_tile):
    M, K = x.shape
    Nout = w.shape[0]
    grid = Nout // n_tile
    return pl.pallas_call(
        functools.partial(_mm_body, relu=relu),
        grid=(grid,),
        in_specs=[
            pl.BlockSpec((M, K), lambda i: (0, 0)),
            pl.BlockSpec((n_tile, K), lambda i: (i, 0)),
            pl.BlockSpec((1, n_tile), lambda i: (0, i)),
        ],
        out_specs=pl.BlockSpec((M, n_tile), lambda i: (0, i)),
        out_shape=jax.ShapeDtypeStruct((M, Nout), jnp.float32),
    )(x, w, b)


def kernel(partial, W_c1, b_c1, bn1_g, bn1_b, W_c2, b_c2, W_c3, b_c3, bn2_g,
           bn2_b, W_c4, b_c4, W_l1, b_l1, W_l2, b_l2, W_l3, b_l3, W_l4, b_l4):
    B, N, _ = partial.shape

    s1 = bn1_g / jnp.sqrt(1.0 + _EPS)
    s2 = bn2_g / jnp.sqrt(1.0 + _EPS)
    w1 = W_c1 * s1[:, None]
    b1 = (b_c1 * s1 + bn1_b).reshape(1, -1)
    w3 = W_c3 * s2[:, None]
    b3 = (b_c3 * s2 + bn2_b).reshape(1, -1)

    xp = jnp.pad(partial, ((0, 0), (0, 0), (0, 5)))
    w1t = jnp.pad(w1, ((0, 0), (0, 5))).T
    w2t = W_c2.T
    wgt = w3[:, :256].T
    wft = w3[:, 256:].T
    w4t = W_c4.T

    z = _encoder(xp, w1t, b1, w2t, b2_r(b_c2), wft, wgt, b3, w4t,
                 b2_r(b_c4))

    d = _mm(z, W_l1, b2_r(b_l1), True, 2048)
    d = _mm(d, W_l2, b2_r(b_l2), True, 1024)
    d = _mm(d, W_l3, b2_r(b_l3), True, 1024)
    d = _mm(d, W_l4, b2_r(b_l4), False, 1024)
    return d.reshape(B, -1, 3)


def b2_r(b):
    return b.reshape(1, -1)


# speedup vs baseline: 1.0287x; 1.0287x over previous
import functools

import jax
import jax.numpy as jnp
from jax.experimental import pallas as pl

_EPS = 1e-5
_S = 2


def _enc_body(xp_ref, w1t_ref, b1_ref, w2t_ref, b2_ref, wft_ref, wgt_ref,
              b3_ref, w4t_ref, b4_ref, z_ref):
    S, N, _ = xp_ref.shape
    x = xp_ref[...].reshape(S * N, 8)
    f1 = jnp.maximum(
        jnp.dot(x, w1t_ref[...], preferred_element_type=jnp.float32)
        + b1_ref[...], 0.0)
    f = (jnp.dot(f1, w2t_ref[...], preferred_element_type=jnp.float32)
         + b2_ref[...])
    g = jnp.max(f.reshape(S, N, 256), axis=1)
    g8 = jnp.broadcast_to(g.reshape(S, 1, 256), (S, 8, 256)).reshape(S * 8, 256)
    gterm = jnp.dot(g8, wgt_ref[...],
                    preferred_element_type=jnp.float32)
    gfull = jnp.broadcast_to(
        gterm.reshape(S, 8, 512)[:, :1, :], (S, N, 512)).reshape(S * N, 512)
    h1 = jnp.maximum(
        jnp.dot(f, wft_ref[...], preferred_element_type=jnp.float32)
        + gfull + b3_ref[...], 0.0)
    h = (jnp.dot(h1, w4t_ref[...], preferred_element_type=jnp.float32)
         + b4_ref[...])
    z_ref[0] = jnp.max(h.reshape(S, N, 1024), axis=1)


def _encoder(xp, w1t, b1, w2t, b2, wft, wgt, b3, w4t, b4):
    B, N, _ = xp.shape
    C = w4t.shape[1]
    rep = lambda shape: pl.BlockSpec(shape, lambda b: (0,) * len(shape))
    return pl.pallas_call(
        _enc_body,
        grid=(B // _S,),
        in_specs=[
            pl.BlockSpec((_S, N, 8), lambda b: (b, 0, 0)),
            rep(w1t.shape), rep(b1.shape), rep(w2t.shape), rep(b2.shape),
            rep(wft.shape), rep(wgt.shape), rep(b3.shape), rep(w4t.shape),
            rep(b4.shape),
        ],
        out_specs=pl.BlockSpec((1, _S, C), lambda b: (b, 0, 0)),
        out_shape=jax.ShapeDtypeStruct((B // _S, _S, C), jnp.float32),
    )(xp, w1t, b1, w2t, b2, wft, wgt, b3, w4t, b4).reshape(B, C)


def _mm_body(x_ref, w_ref, b_ref, o_ref, *, relu):
    y = jax.lax.dot_general(
        x_ref[...], w_ref[...], (((1,), (1,)), ((), ())),
        preferred_element_type=jnp.float32) + b_ref[...]
    if relu:
        y = jnp.maximum(y, 0.0)
    o_ref[...] = y


def _mm(x, w, b, relu, n_tile):
    M, K = x.shape
    Nout = w.shape[0]
    grid = Nout // n_tile
    return pl.pallas_call(
        functools.partial(_mm_body, relu=relu),
        grid=(grid,),
        in_specs=[
            pl.BlockSpec((M, K), lambda i: (0, 0)),
            pl.BlockSpec((n_tile, K), lambda i: (i, 0)),
            pl.BlockSpec((1, n_tile), lambda i: (0, i)),
        ],
        out_specs=pl.BlockSpec((M, n_tile), lambda i: (0, i)),
        out_shape=jax.ShapeDtypeStruct((M, Nout), jnp.float32),
    )(x, w, b)


def kernel(partial, W_c1, b_c1, bn1_g, bn1_b, W_c2, b_c2, W_c3, b_c3, bn2_g,
           bn2_b, W_c4, b_c4, W_l1, b_l1, W_l2, b_l2, W_l3, b_l3, W_l4, b_l4):
    B, N, _ = partial.shape

    s1 = bn1_g / jnp.sqrt(1.0 + _EPS)
    s2 = bn2_g / jnp.sqrt(1.0 + _EPS)
    w1 = W_c1 * s1[:, None]
    b1 = (b_c1 * s1 + bn1_b).reshape(1, -1)
    w3 = W_c3 * s2[:, None]
    b3 = (b_c3 * s2 + bn2_b).reshape(1, -1)

    xp = jnp.pad(partial, ((0, 0), (0, 0), (0, 5)))
    w1t = jnp.pad(w1, ((0, 0), (0, 5))).T
    w2t = W_c2.T
    wgt = w3[:, :256].T
    wft = w3[:, 256:].T
    w4t = W_c4.T

    z = _encoder(xp, w1t, b1, w2t, b2_r(b_c2), wft, wgt, b3, w4t,
                 b2_r(b_c4))

    d = _mm(z, W_l1, b2_r(b_l1), True, 2048)
    d = _mm(d, W_l2, b2_r(b_l2), True, 1024)
    d = _mm(d, W_l3, b2_r(b_l3), True, 1024)
    d = _mm(d, W_l4, b2_r(b_l4), False, 1024)
    return d.reshape(B, -1, 3)


def b2_r(b):
    return b.reshape(1, -1)
